# baseline (speedup 1.0000x reference)
import functools

import jax
import jax.numpy as jnp
from jax.experimental import pallas as pl
from jax.experimental.pallas import tpu as pltpu

STEP = 16
LEAK = 32
KERNEL_SIZE = STEP + LEAK
PADDING = KERNEL_SIZE
FODEP = KERNEL_SIZE
SYNAPSES = 256
NEURONS = 64
IN_CH = 1
OUT_CH = 10
DENSE = 0.05
THETA = DENSE * (SYNAPSES * IN_CH)
BIAS = 0.5
BATCH = 16
TIME = 128
T_OUT = TIME + 2 * PADDING - KERNEL_SIZE + 1
T_PAD = TIME + 2 * PADDING


def _conv_body(xt_ref, wt_ref, out_ref):
    out_ref[...] = jnp.zeros_like(out_ref)
    w = wt_ref[...]
    for k in range(KERNEL_SIZE):
        t = float(KERNEL_SIZE - 1 - k)
        t_spike = jnp.float32(t / STEP)
        t_leak = -(t - w * STEP) / LEAK + w
        tap = jnp.maximum(0.0, jnp.minimum(t_spike, t_leak))
        for b in range(BATCH):
            xkb = xt_ref[b, pl.ds(k, T_OUT), :]
            out_ref[b] += jnp.dot(xkb, tap, preferred_element_type=jnp.float32)
    out_ref[...] = out_ref[...] + jnp.float32(BIAS * THETA)


@jax.jit
def _potentials(input_spikes, weight):
    x = input_spikes.reshape(BATCH, IN_CH * SYNAPSES, TIME)
    xt = jnp.transpose(x, (0, 2, 1))
    xt = jnp.pad(xt, ((0, 0), (PADDING, PADDING), (0, 0)))
    wt = jnp.transpose(weight, (1, 0))
    pots = pl.pallas_call(
        _conv_body,
        out_shape=jax.ShapeDtypeStruct((BATCH, T_OUT, OUT_CH * NEURONS), jnp.float32),
    )(xt, wt)
    return pots


def kernel(input_spikes, weight):
    pots = _potentials(input_spikes, weight)
    pots = pots.reshape(BATCH, T_OUT, OUT_CH, NEURONS)
    pots_t = jnp.transpose(pots, (1, 0, 3, 2))
    dep0 = jnp.zeros((BATCH, NEURONS, OUT_CH), dtype=jnp.float32)

    def body(dep, pot_t):
        pot_t = pot_t * (dep == 0).astype(pot_t.dtype)
        winner = jnp.argmax(pot_t, axis=-1)
        gathered = jnp.take_along_axis(pot_t, winner[..., None], axis=-1)
        spike = (gathered > THETA).astype(pot_t.dtype)
        winners_t = jax.nn.one_hot(winner, OUT_CH, dtype=pot_t.dtype) * spike
        dep = dep + jnp.sum(winners_t, axis=-1, keepdims=True) * FODEP
        dep = jnp.clip(dep - 1.0, 0.0, FODEP - 1.0)
        return dep, winners_t

    _, winners = jax.lax.scan(body, dep0, pots_t)
    return jnp.transpose(winners, (1, 3, 2, 0)).astype(jnp.float32)


# speedup vs baseline: 29.8847x; 29.8847x over previous
import functools

import jax
import jax.numpy as jnp
from jax import lax
from jax.experimental import pallas as pl
from jax.experimental.pallas import tpu as pltpu
from jax.experimental.pallas import tpu_sc as plsc

STEP = 16
LEAK = 32
KERNEL_SIZE = STEP + LEAK
PADDING = KERNEL_SIZE
FODEP = KERNEL_SIZE
SYNAPSES = 256
NEURONS = 64
IN_CH = 1
OUT_CH = 10
DENSE = 0.05
THETA = DENSE * (SYNAPSES * IN_CH)
BIAS = 0.5
BATCH = 16
TIME = 128
T_OUT = TIME + 2 * PADDING - KERNEL_SIZE + 1
T_PAD = TIME + 2 * PADDING
TP = 192
ROWS = BATCH * NEURONS
NC = 2
NS = 16
NW = NC * NS
ROWS_PER_W = ROWS // NW
NCHUNK = TP // 16
INF = 1 << 24
MAX_SPIKES = (T_OUT + FODEP - 1) // FODEP


def _conv_body(xp_ref, w_ref, out_ref):
    out_ref[...] = jnp.zeros_like(out_ref)
    w = w_ref[...]
    for k in range(KERNEL_SIZE):
        t = float(KERNEL_SIZE - 1 - k)
        t_spike = jnp.float32(t / STEP)
        t_leak = -(t - w * STEP) / LEAK + w
        tap = jnp.maximum(0.0, jnp.minimum(t_spike, t_leak))
        for b in range(BATCH):
            xkb = xp_ref[b, :, pl.ds(k, T_OUT)]
            out_ref[b, :, pl.ds(0, T_OUT)] += jnp.dot(
                tap, xkb, preferred_element_type=jnp.float32)
    out_ref[...] = out_ref[...] + jnp.float32(BIAS * THETA)


def _wta_body(pots_hbm, out_hbm, row_v, stage_v, nc_v):
    theta = jnp.float32(THETA)
    wid = lax.axis_index("s") * NC + lax.axis_index("c")
    lane = jax.lax.iota(jnp.int32, 16)
    lane0 = lane == 0
    ones16f = jnp.full((16,), 1.0, jnp.float32)
    zeros16f = jnp.zeros((16,), jnp.float32)
    inf16f = jnp.full((16,), float(INF), jnp.float32)
    c_idx = jnp.minimum(lane, OUT_CH - 1)

    gdn = lax.GatherDimensionNumbers(
        offset_dims=(), collapsed_slice_dims=(0,), start_index_map=(0,))

    def vgather(v, idx):
        return lax.gather(v, idx[:, None], gdn, (1,),
                          mode=lax.GatherScatterMode.PROMISE_IN_BOUNDS)

    def suffix_min(v):
        for d in (1, 2, 4, 8):
            v = jnp.minimum(v, vgather(v, jnp.minimum(lane + d, 15)))
        return v

    def bcast0(v):
        return vgather(v, jnp.zeros((16,), jnp.int32))

    def allmax(v):
        for d in (1, 2, 4, 8):
            v = jnp.maximum(v, vgather(v, lane ^ d))
        return v

    for c in range(OUT_CH):
        for j in range(NCHUNK):
            stage_v[c, pl.ds(j * 16, 16)] = zeros16f

    def do_row(i, _):
        row = wid * ROWS_PER_W + i
        b = row // NEURONS
        n = row % NEURONS
        for c in range(OUT_CH):
            pltpu.sync_copy(pots_hbm.at[b, c * NEURONS + n], row_v.at[c])

        carry = inf16f
        for j in range(NCHUNK - 1, -1, -1):
            m = row_v[0, pl.ds(j * 16, 16)]
            for c in range(1, OUT_CH):
                m = jnp.maximum(m, row_v[c, pl.ds(j * 16, 16)])
            tg = (lane + j * 16).astype(jnp.float32)
            val = jnp.where((m > theta) & (tg <= float(T_OUT - 1)), tg,
                            jnp.float32(INF))
            nc_chunk = jnp.minimum(suffix_min(val), carry)
            nc_v[pl.ds(j * 16, 16)] = nc_chunk
            carry = bcast0(nc_chunk)

        t = carry
        spikes = []
        for _s in range(MAX_SPIKES):
            active = t <= float(T_OUT - 1)
            t_safe = jnp.minimum(t, float(TP - 1)).astype(jnp.int32)
            g = plsc.load_gather(row_v, [c_idx, t_safe])
            mval = allmax(g)
            win = plsc.all_reduce_ffs(g == mval)
            win_vec = jnp.minimum(jnp.full((16,), 1, jnp.int32) * win, OUT_CH - 1)
            plsc.store_scatter(
                stage_v, [win_vec, t_safe], ones16f, mask=lane0 & active)
            spikes.append((win_vec, t_safe, active))
            tt = jnp.minimum(t + FODEP, float(TP - 1)).astype(jnp.int32)
            nxt_v = plsc.load_gather(nc_v, [tt])
            t = jnp.where(t + FODEP > T_OUT - 1, inf16f, nxt_v)

        for c in range(OUT_CH):
            pltpu.sync_copy(stage_v.at[c], out_hbm.at[b, c, n])

        for win_vec, t_safe, active in spikes:
            plsc.store_scatter(
                stage_v, [win_vec, t_safe], zeros16f, mask=lane0 & active)

        return 0

    lax.fori_loop(0, ROWS_PER_W, do_row, 0)


@functools.partial(
    pl.kernel,
    mesh=plsc.VectorSubcoreMesh(core_axis_name="c", subcore_axis_name="s"),
    out_type=jax.ShapeDtypeStruct((BATCH, OUT_CH, NEURONS, TP), jnp.float32),
    scratch_types=[
        pltpu.VMEM((OUT_CH, TP), jnp.float32),
        pltpu.VMEM((OUT_CH, TP), jnp.float32),
        pltpu.VMEM((TP,), jnp.float32),
    ],
    compiler_params=pltpu.CompilerParams(needs_layout_passes=False),
)
def _wta(pots_hbm, out_hbm, row_v, stage_v, nc_v):
    _wta_body(pots_hbm, out_hbm, row_v, stage_v, nc_v)


@jax.jit
def kernel(input_spikes, weight):
    x = input_spikes.reshape(BATCH, IN_CH * SYNAPSES, TIME)
    xp = jnp.pad(x, ((0, 0), (0, 0), (PADDING, PADDING)))
    pots = pl.pallas_call(
        _conv_body,
        out_shape=jax.ShapeDtypeStruct((BATCH, OUT_CH * NEURONS, TP), jnp.float32),
    )(xp, weight)
    return _wta(pots)[..., :T_OUT]


# speedup vs baseline: 69.6122x; 2.3294x over previous
import functools

import jax
import jax.numpy as jnp
from jax import lax
from jax.experimental import pallas as pl
from jax.experimental.pallas import tpu as pltpu
from jax.experimental.pallas import tpu_sc as plsc

STEP = 16
LEAK = 32
KERNEL_SIZE = STEP + LEAK
PADDING = KERNEL_SIZE
FODEP = KERNEL_SIZE
SYNAPSES = 256
NEURONS = 64
IN_CH = 1
OUT_CH = 10
DENSE = 0.05
THETA = DENSE * (SYNAPSES * IN_CH)
BIAS = 0.5
BATCH = 16
TIME = 128
T_OUT = TIME + 2 * PADDING - KERNEL_SIZE + 1
T_PAD = TIME + 2 * PADDING
TP = 192
ROWS = BATCH * NEURONS
NC = 2
NS = 16
NW = NC * NS
ROWS_PER_W = ROWS // NW
NCHUNK = TP // 16
INF = 1 << 24
MAX_SPIKES = (T_OUT + FODEP - 1) // FODEP


def _conv_body(xp_ref, w_ref, out_ref):
    out_ref[...] = jnp.zeros_like(out_ref)
    w = w_ref[...]
    for k in range(KERNEL_SIZE):
        t = float(KERNEL_SIZE - 1 - k)
        t_spike = jnp.float32(t / STEP)
        t_leak = -(t - w * STEP) / LEAK + w
        tap = jnp.maximum(0.0, jnp.minimum(t_spike, t_leak))
        for b in range(BATCH):
            xkb = xp_ref[b, :, pl.ds(k, T_OUT)]
            out_ref[b, :, pl.ds(0, T_OUT)] += jnp.dot(
                tap, xkb, preferred_element_type=jnp.float32)
    out_ref[...] = out_ref[...] + jnp.float32(BIAS * THETA)


def _wta_body(pots_hbm, out_hbm, row_v, row2_v, stage_v, stage2_v, nc_v,
              semi0, semi1, semo0, semo1):
    theta = jnp.float32(THETA)
    wid = lax.axis_index("s") * NC + lax.axis_index("c")
    lane = jax.lax.iota(jnp.int32, 16)
    lane0 = lane == 0
    ones16f = jnp.full((16,), 1.0, jnp.float32)
    zeros16f = jnp.zeros((16,), jnp.float32)
    inf16f = jnp.full((16,), float(INF), jnp.float32)
    c_idx = jnp.minimum(lane, OUT_CH - 1)

    gdn = lax.GatherDimensionNumbers(
        offset_dims=(), collapsed_slice_dims=(0,), start_index_map=(0,))

    def vgather(v, idx):
        return lax.gather(v, idx[:, None], gdn, (1,),
                          mode=lax.GatherScatterMode.PROMISE_IN_BOUNDS)

    def suffix_min(v):
        for d in (1, 2, 4, 8):
            v = jnp.minimum(v, vgather(v, jnp.minimum(lane + d, 15)))
        return v

    def bcast0(v):
        return vgather(v, jnp.zeros((16,), jnp.int32))

    def allmax(v):
        for d in (1, 2, 4, 8):
            v = jnp.maximum(v, vgather(v, lane ^ d))
        return v

    def bn_of(r):
        return r // NEURONS, r % NEURONS

    def issue_in(r, rbuf, sem):
        b, n = bn_of(jnp.minimum(r, ROWS - 1))
        for c in range(OUT_CH):
            pltpu.async_copy(pots_hbm.at[b, c * NEURONS + n], rbuf.at[c], sem)

    def drain_in(r, rbuf, sem):
        b, n = bn_of(jnp.minimum(r, ROWS - 1))
        for c in range(OUT_CH):
            pltpu.make_async_copy(
                pots_hbm.at[b, c * NEURONS + n], rbuf.at[c], sem).wait()

    def issue_out(r, sbuf, sem):
        b, n = bn_of(r)
        for c in range(OUT_CH):
            pltpu.async_copy(sbuf.at[c], out_hbm.at[b, c, n], sem)

    def drain_out(r, sbuf, sem):
        b, n = bn_of(r)
        for c in range(OUT_CH):
            pltpu.make_async_copy(
                sbuf.at[c], out_hbm.at[b, c, n], sem).wait()

    def process(rbuf, sbuf):
        carry = inf16f
        for j in range(NCHUNK - 1, -1, -1):
            m = rbuf[0, pl.ds(j * 16, 16)]
            for c in range(1, OUT_CH):
                m = jnp.maximum(m, rbuf[c, pl.ds(j * 16, 16)])
            tg = (lane + j * 16).astype(jnp.float32)
            val = jnp.where((m > theta) & (tg <= float(T_OUT - 1)), tg,
                            jnp.float32(INF))
            nc_chunk = jnp.minimum(suffix_min(val), carry)
            nc_v[pl.ds(j * 16, 16)] = nc_chunk
            carry = bcast0(nc_chunk)

        t = carry
        for _s in range(MAX_SPIKES):
            active = t <= float(T_OUT - 1)
            t_safe = jnp.minimum(t, float(TP - 1)).astype(jnp.int32)
            g = plsc.load_gather(rbuf, [c_idx, t_safe])
            mval = allmax(g)
            win = plsc.all_reduce_ffs(g == mval)
            win_vec = jnp.minimum(jnp.full((16,), 1, jnp.int32) * win, OUT_CH - 1)
            plsc.store_scatter(
                sbuf, [win_vec, t_safe], ones16f, mask=lane0 & active)
            tt = jnp.minimum(t + FODEP, float(TP - 1)).astype(jnp.int32)
            nxt_v = plsc.load_gather(nc_v, [tt])
            t = jnp.where(t + FODEP > T_OUT - 1, inf16f, nxt_v)

    def zero_stage(sbuf):
        for c in range(OUT_CH):
            for j in range(NCHUNK):
                sbuf[c, pl.ds(j * 16, 16)] = zeros16f

    zero_stage(stage_v)
    zero_stage(stage2_v)
    row0 = wid * ROWS_PER_W
    issue_in(row0, row_v, semi0)
    issue_in(row0 + 1, row2_v, semi1)

    def do_pair(p, _):
        for par, rbuf, sbuf, semi, semo in (
            (0, row_v, stage_v, semi0, semo0),
            (1, row2_v, stage2_v, semi1, semo1),
        ):
            r = row0 + 2 * p + par
            drain_in(r, rbuf, semi)

            @pl.when(p >= 1)
            def _():
                drain_out(r - 2, sbuf, semo)
                zero_stage(sbuf)

            process(rbuf, sbuf)
            issue_out(r, sbuf, semo)
            issue_in(r + 2, rbuf, semi)
        return 0

    lax.fori_loop(0, ROWS_PER_W // 2, do_pair, 0)

    last = row0 + ROWS_PER_W
    drain_out(last - 2, stage_v, semo0)
    drain_out(last - 1, stage2_v, semo1)
    drain_in(last, row_v, semi0)
    drain_in(last + 1, row2_v, semi1)


@functools.partial(
    pl.kernel,
    mesh=plsc.VectorSubcoreMesh(core_axis_name="c", subcore_axis_name="s"),
    out_type=jax.ShapeDtypeStruct((BATCH, OUT_CH, NEURONS, TP), jnp.float32),
    scratch_types=[
        pltpu.VMEM((OUT_CH, TP), jnp.float32),
        pltpu.VMEM((OUT_CH, TP), jnp.float32),
        pltpu.VMEM((OUT_CH, TP), jnp.float32),
        pltpu.VMEM((OUT_CH, TP), jnp.float32),
        pltpu.VMEM((TP,), jnp.float32),
        pltpu.SemaphoreType.DMA,
        pltpu.SemaphoreType.DMA,
        pltpu.SemaphoreType.DMA,
        pltpu.SemaphoreType.DMA,
    ],
    compiler_params=pltpu.CompilerParams(needs_layout_passes=False),
)
def _wta(pots_hbm, out_hbm, row_v, row2_v, stage_v, stage2_v, nc_v,
         semi0, semi1, semo0, semo1):
    _wta_body(pots_hbm, out_hbm, row_v, row2_v, stage_v, stage2_v, nc_v,
              semi0, semi1, semo0, semo1)


@jax.jit
def kernel(input_spikes, weight):
    x = input_spikes.reshape(BATCH, IN_CH * SYNAPSES, TIME)
    xp = jnp.pad(x, ((0, 0), (0, 0), (PADDING, PADDING)))
    pots = pl.pallas_call(
        _conv_body,
        out_shape=jax.ShapeDtypeStruct((BATCH, OUT_CH * NEURONS, TP), jnp.float32),
    )(xp, weight)
    return _wta(pots)[..., :T_OUT]


# speedup vs baseline: 77.0559x; 1.1069x over previous
import functools

import jax
import jax.numpy as jnp
from jax import lax
from jax.experimental import pallas as pl
from jax.experimental.pallas import tpu as pltpu
from jax.experimental.pallas import tpu_sc as plsc

STEP = 16
LEAK = 32
KERNEL_SIZE = STEP + LEAK
PADDING = KERNEL_SIZE
FODEP = KERNEL_SIZE
SYNAPSES = 256
NEURONS = 64
IN_CH = 1
OUT_CH = 10
DENSE = 0.05
THETA = DENSE * (SYNAPSES * IN_CH)
BIAS = 0.5
BATCH = 16
TIME = 128
T_OUT = TIME + 2 * PADDING - KERNEL_SIZE + 1
T_PAD = TIME + 2 * PADDING
TP = 192
ROWS = BATCH * NEURONS
NC = 2
NS = 16
NW = NC * NS
ROWS_PER_W = ROWS // NW
NCHUNK = TP // 16
INF = 1 << 24
MAX_SPIKES = (T_OUT + FODEP - 1) // FODEP


def _conv_body(x_ref, w_ref, out_ref):
    out_ref[...] = jnp.zeros_like(out_ref)
    w = w_ref[...]
    for k in range(KERNEL_SIZE - 1):
        t = float(KERNEL_SIZE - 1 - k)
        t_spike = jnp.float32(t / STEP)
        t_leak = -(t - w * STEP) / LEAK + w
        tap = jnp.maximum(0.0, jnp.minimum(t_spike, t_leak))
        for b in range(BATCH):
            out_ref[b, :, pl.ds(PADDING - k, TIME)] += jnp.dot(
                tap, x_ref[b], preferred_element_type=jnp.float32)
    out_ref[...] = out_ref[...] + jnp.float32(BIAS * THETA)


def _wta_body(pots_hbm, out_hbm, row_v, row2_v, stage_v, stage2_v, nc_v,
              semi0, semi1, semo0, semo1):
    theta = jnp.float32(THETA)
    wid = lax.axis_index("s") * NC + lax.axis_index("c")
    lane = jax.lax.iota(jnp.int32, 16)
    lane0 = lane == 0
    ones16f = jnp.full((16,), 1.0, jnp.float32)
    zeros16f = jnp.zeros((16,), jnp.float32)
    inf16f = jnp.full((16,), float(INF), jnp.float32)
    c_idx = jnp.minimum(lane, OUT_CH - 1)

    gdn = lax.GatherDimensionNumbers(
        offset_dims=(), collapsed_slice_dims=(0,), start_index_map=(0,))

    def vgather(v, idx):
        return lax.gather(v, idx[:, None], gdn, (1,),
                          mode=lax.GatherScatterMode.PROMISE_IN_BOUNDS)

    def suffix_min(v):
        for d in (1, 2, 4, 8):
            v = jnp.minimum(v, vgather(v, jnp.minimum(lane + d, 15)))
        return v

    def bcast0(v):
        return vgather(v, jnp.zeros((16,), jnp.int32))

    def allmax(v):
        for d in (1, 2, 4, 8):
            v = jnp.maximum(v, vgather(v, lane ^ d))
        return v

    def bn_of(r):
        return r // NEURONS, r % NEURONS

    def issue_in(r, rbuf, sem):
        b, n = bn_of(jnp.minimum(r, ROWS - 1))
        for c in range(OUT_CH):
            pltpu.async_copy(pots_hbm.at[b, c * NEURONS + n], rbuf.at[c], sem)

    def drain_in(r, rbuf, sem):
        b, n = bn_of(jnp.minimum(r, ROWS - 1))
        for c in range(OUT_CH):
            pltpu.make_async_copy(
                pots_hbm.at[b, c * NEURONS + n], rbuf.at[c], sem).wait()

    def issue_out(r, sbuf, sem):
        b, n = bn_of(r)
        for c in range(OUT_CH):
            pltpu.async_copy(sbuf.at[c], out_hbm.at[b, c, n], sem)

    def drain_out(r, sbuf, sem):
        b, n = bn_of(r)
        for c in range(OUT_CH):
            pltpu.make_async_copy(
                sbuf.at[c], out_hbm.at[b, c, n], sem).wait()

    def process(rbuf, sbuf):
        carry = inf16f
        for j in range(NCHUNK - 1, -1, -1):
            m = rbuf[0, pl.ds(j * 16, 16)]
            for c in range(1, OUT_CH):
                m = jnp.maximum(m, rbuf[c, pl.ds(j * 16, 16)])
            tg = (lane + j * 16).astype(jnp.float32)
            val = jnp.where((m > theta) & (tg <= float(T_OUT - 1)), tg,
                            jnp.float32(INF))
            nc_chunk = jnp.minimum(suffix_min(val), carry)
            nc_v[pl.ds(j * 16, 16)] = nc_chunk
            carry = bcast0(nc_chunk)

        t = carry
        for _s in range(MAX_SPIKES):
            active = t <= float(T_OUT - 1)
            t_safe = jnp.minimum(t, float(TP - 1)).astype(jnp.int32)
            g = plsc.load_gather(rbuf, [c_idx, t_safe])
            mval = allmax(g)
            win = plsc.all_reduce_ffs(g == mval)
            win_vec = jnp.minimum(jnp.full((16,), 1, jnp.int32) * win, OUT_CH - 1)
            plsc.store_scatter(
                sbuf, [win_vec, t_safe], ones16f, mask=lane0 & active)
            tt = jnp.minimum(t + FODEP, float(TP - 1)).astype(jnp.int32)
            nxt_v = plsc.load_gather(nc_v, [tt])
            t = jnp.where(t + FODEP > T_OUT - 1, inf16f, nxt_v)

    def zero_stage(sbuf):
        for c in range(OUT_CH):
            for j in range(NCHUNK):
                sbuf[c, pl.ds(j * 16, 16)] = zeros16f

    zero_stage(stage_v)
    zero_stage(stage2_v)
    row0 = wid * ROWS_PER_W
    issue_in(row0, row_v, semi0)
    issue_in(row0 + 1, row2_v, semi1)

    def do_pair(p, _):
        for par, rbuf, sbuf, semi, semo in (
            (0, row_v, stage_v, semi0, semo0),
            (1, row2_v, stage2_v, semi1, semo1),
        ):
            r = row0 + 2 * p + par
            drain_in(r, rbuf, semi)

            @pl.when(p >= 1)
            def _():
                drain_out(r - 2, sbuf, semo)
                zero_stage(sbuf)

            process(rbuf, sbuf)
            issue_out(r, sbuf, semo)
            issue_in(r + 2, rbuf, semi)
        return 0

    lax.fori_loop(0, ROWS_PER_W // 2, do_pair, 0)

    last = row0 + ROWS_PER_W
    drain_out(last - 2, stage_v, semo0)
    drain_out(last - 1, stage2_v, semo1)
    drain_in(last, row_v, semi0)
    drain_in(last + 1, row2_v, semi1)


@functools.partial(
    pl.kernel,
    mesh=plsc.VectorSubcoreMesh(core_axis_name="c", subcore_axis_name="s"),
    out_type=jax.ShapeDtypeStruct((BATCH, OUT_CH, NEURONS, TP), jnp.float32),
    scratch_types=[
        pltpu.VMEM((OUT_CH, TP), jnp.float32),
        pltpu.VMEM((OUT_CH, TP), jnp.float32),
        pltpu.VMEM((OUT_CH, TP), jnp.float32),
        pltpu.VMEM((OUT_CH, TP), jnp.float32),
        pltpu.VMEM((TP,), jnp.float32),
        pltpu.SemaphoreType.DMA,
        pltpu.SemaphoreType.DMA,
        pltpu.SemaphoreType.DMA,
        pltpu.SemaphoreType.DMA,
    ],
    compiler_params=pltpu.CompilerParams(needs_layout_passes=False),
)
def _wta(pots_hbm, out_hbm, row_v, row2_v, stage_v, stage2_v, nc_v,
         semi0, semi1, semo0, semo1):
    _wta_body(pots_hbm, out_hbm, row_v, row2_v, stage_v, stage2_v, nc_v,
              semi0, semi1, semo0, semo1)


@jax.jit
def kernel(input_spikes, weight):
    x = input_spikes.reshape(BATCH, IN_CH * SYNAPSES, TIME)
    pots = pl.pallas_call(
        _conv_body,
        out_shape=jax.ShapeDtypeStruct((BATCH, OUT_CH * NEURONS, TP), jnp.float32),
    )(x, weight)
    return _wta(pots)[..., :T_OUT]
